# baseline (speedup 1.0000x reference)
import jax
import jax.numpy as jnp
import numpy as np
from jax.experimental import pallas as pl

B, T, J, F = 64, 150, 25, 3
H, NH, FEATD, NL, OUT, RED = 64, 4, 300, 2, 60, 16


def _ln(x, g, b):
    m = x.mean(-1, keepdims=True)
    v = ((x - m) ** 2).mean(-1, keepdims=True)
    return (x - m) / jnp.sqrt(v + 1e-5) * g + b


def _sgcn(x, W, b):
    return jnp.transpose(x @ W + b, (0, 3, 1, 2))


def _gcn(x, ei, W, b):
    N = x.shape[0]
    sl = jnp.arange(N)
    src = jnp.concatenate([ei[0], sl])
    dst = jnp.concatenate([ei[1], sl])
    deg = jnp.zeros((N,), x.dtype).at[dst].add(1.0)
    dinv = 1.0 / jnp.sqrt(deg)
    norm = dinv[src] * dinv[dst]
    h = x @ W
    msg = h[src] * norm[:, None]
    return jnp.zeros((N, h.shape[1]), x.dtype).at[dst].add(msg) + b


def _forward(x, edge_index, p):
    relu = jax.nn.relu
    sk = relu(_sgcn(x, p['sk1_W'], p['sk1_b'])).transpose(0, 2, 3, 1)
    sk = relu(_sgcn(sk, p['sk2_W'], p['sk2_b'])).transpose(0, 2, 3, 1)
    sk = sk.reshape(B * T, J, H)
    hi = sk.transpose(0, 2, 1)[..., None]
    s = hi.mean(axis=(2, 3))
    s = jax.nn.sigmoid(jax.nn.relu(s @ p['se_W1'] + p['se_b1']) @ p['se_W2'] + p['se_b2'])
    sk = (hi * s[:, :, None, None])[..., 0].transpose(0, 2, 1)
    ti = sk.transpose(1, 0, 2)
    S, NB, D = ti.shape
    qkv = ti @ p['attn_Wqkv'].T + p['attn_bqkv']
    q, k, v = jnp.split(qkv, 3, axis=-1)
    dh = D // NH
    def rs(t):
        return t.reshape(S, NB, NH, dh).transpose(1, 2, 0, 3)
    q, k, v = rs(q), rs(k), rs(v)
    att = jax.nn.softmax(q @ k.transpose(0, 1, 3, 2) / np.float32(np.sqrt(dh)), axis=-1)
    ao = (att @ v).transpose(2, 0, 1, 3).reshape(S, NB, D) @ p['attn_Wo'].T + p['attn_bo']
    h1 = _ln(ti + ao, p['ln1_g'], p['ln1_b'])
    ff = relu(h1 @ p['ff_W1'] + p['ff_b1']) @ p['ff_W2'] + p['ff_b2']
    to = _ln(h1 + ff, p['ln2_g'], p['ln2_b'])
    skf = to.transpose(1, 0, 2).reshape(B, T, J * H)
    pos = x[..., :3]
    diff = pos[:, :, :, None, :] - pos[:, :, None, :, :]
    proj = diff @ p['mah_L']
    md = jnp.sqrt(jnp.sum(proj ** 2, -1) + 1e-12)
    iu, ju = np.triu_indices(J, 1)
    jcd = md[:, :, iu, ju]
    jf = jcd[:, :, None, :]
    jf = relu(_sgcn(jf, p['jcd1_W'], p['jcd1_b'])).transpose(0, 2, 3, 1)
    jf = relu(_sgcn(jf, p['jcd2_W'], p['jcd2_b'])).transpose(0, 2, 3, 1)
    jcdf = jf.reshape(B, T, H)
    sp = relu(_sgcn(x, p['sp1_W'], p['sp1_b'])).transpose(0, 2, 3, 1)
    sp = relu(_sgcn(sp, p['sp2_W'], p['sp2_b'])).transpose(0, 2, 3, 1)
    spf = sp.reshape(B, T, J, H).mean(2)
    dist = jnp.sqrt(jnp.sum(diff ** 2, -1) + 1e-12)
    feats = []
    for kk in [2, 3, 4]:
        ak = min(kk, J - 1)
        nv, _ = jax.lax.top_k(-dist, ak + 1)
        kd = -nv[..., 1:ak + 1]
        avg = kd.mean(-1)
        sd = kd.std(-1, ddof=1)
        mn = kd.min(-1)
        feats.append(jnp.stack([avg, sd, mn], -1).mean(1))
    topo = jnp.concatenate(feats, -1).reshape(B * J, 9)
    tei = edge_index % J
    t1 = relu(_gcn(topo, tei, p['topo1_W'], p['topo1_b']))
    t2 = relu(_gcn(t1, tei, p['topo2_W'], p['topo2_b']))
    topof = jnp.broadcast_to(t2.reshape(B, J, H).mean(1)[:, None, :], (B, T, H))
    mf = x.mean(1); sdf = x.std(1, ddof=1); mnf = x.min(1); mxf = x.max(1)
    cen = x - mf[:, None]
    var = x.var(1, ddof=1) + 1e-10
    skw = (cen ** 3).mean(1) / (jnp.sqrt(var) ** 3 + 1e-10)
    kur = (cen ** 4).mean(1) / (var ** 2 + 1e-10) - 3.0
    st = jnp.stack([mf.mean(-1), sdf.mean(-1), mnf.mean(-1), mxf.mean(-1), skw.mean(-1), kur.mean(-1)], -1).reshape(B * J, 6)
    s1 = relu(_gcn(st, tei, p['stat1_W'], p['stat1_b']))
    s2 = relu(_gcn(s1, tei, p['stat2_W'], p['stat2_b']))
    stf = jnp.broadcast_to(s2.mean(0)[None, None, :], (B, T, H))
    comb = jnp.concatenate([skf, jcdf, spf, topof, stf], -1)
    fu = relu(_ln(comb @ p['fu1_W'] + p['fu1_b'], p['fln_g'], p['fln_b'])) @ p['fu2_W'] + p['fu2_b']
    h = fu
    for l in range(NL):
        Wih = p['gru%d_Wih' % l]; Whh = p['gru%d_Whh' % l]
        bih = p['gru%d_bih' % l]; bhh = p['gru%d_bhh' % l]
        def step(hh, xt):
            gi = xt @ Wih.T + bih
            gh = hh @ Whh.T + bhh
            ir, iz, inn = jnp.split(gi, 3, -1)
            hr, hz, hn = jnp.split(gh, 3, -1)
            r = jax.nn.sigmoid(ir + hr)
            z = jax.nn.sigmoid(iz + hz)
            n = jnp.tanh(inn + r * hn)
            hh2 = (1 - z) * n + z * hh
            return hh2, hh2
        h0 = jnp.zeros((B, Whh.shape[1]), x.dtype)
        _, ys = jax.lax.scan(step, h0, h.transpose(1, 0, 2))
        h = ys.transpose(1, 0, 2)
    return h[:, -1, :] @ p['fc_W'] + p['fc_b']


def _identity_kernel(x_ref, o_ref):
    o_ref[...] = x_ref[...]


def kernel(x, edge_index, params):
    out = _forward(x, edge_index, params)
    return pl.pallas_call(
        _identity_kernel,
        out_shape=jax.ShapeDtypeStruct(out.shape, out.dtype),
    )(out)


# speedup vs baseline: 3.0058x; 3.0058x over previous
import functools
import numpy as np

import jax
import jax.numpy as jnp
from jax import lax
from jax.experimental import pallas as pl
from jax.experimental.pallas import tpu as pltpu

B, T, J, Fd = 64, 150, 25, 3
H, NH, FEATD, NL, OUT, RED = 64, 4, 300, 2, 60, 16
JP = 32
TBt = 75
SPB = T // TBt
DH = H // NH
NEG = -1e30


def _ln(x, g, b):
    m = jnp.mean(x, axis=-1, keepdims=True)
    v = jnp.mean((x - m) ** 2, axis=-1, keepdims=True)
    return (x - m) * jax.lax.rsqrt(v + 1e-5) * g + b


def _kernel_a(x_ref,
              sk1_W, sk1_b, sk2_W, sk2_b,
              se_W1, se_b1, se_W2, se_b2,
              WqkvT, bqkv, WoT, bo,
              ln1_g, ln1_b, ln2_g, ln2_b,
              ff_W1, ff_b1, ff_W2, ff_b2,
              sp1_W, sp1_b, sp2_W, sp2_b,
              mah_L,
              Wsk_fu, Wfull_jcd, jcd1_b, jcd2_W, jcd2_b, Wjcd_fu, Wsp_fu,
              pf_ref, topo9_ref):
    (sk1_W, sk1_b, sk2_W, sk2_b, se_W1, se_b1, se_W2, se_b2,
     WqkvT, bqkv, WoT, bo, ln1_g, ln1_b, ln2_g, ln2_b,
     ff_W1, ff_b1, ff_W2, ff_b2, sp1_W, sp1_b, sp2_W, sp2_b, mah_L,
     Wsk_fu, Wfull_jcd, jcd1_b, jcd2_W, jcd2_b, Wjcd_fu, Wsp_fu) = [
        r[...] for r in (
            sk1_W, sk1_b, sk2_W, sk2_b, se_W1, se_b1, se_W2, se_b2,
            WqkvT, bqkv, WoT, bo, ln1_g, ln1_b, ln2_g, ln2_b,
            ff_W1, ff_b1, ff_W2, ff_b2, sp1_W, sp1_b, sp2_W, sp2_b, mah_L,
            Wsk_fu, Wfull_jcd, jcd1_b, jcd2_W, jcd2_b, Wjcd_fu, Wsp_fu)]
    X = x_ref[0]
    R = TBt * JP
    f32 = jnp.float32
    dot = functools.partial(lax.dot_general, preferred_element_type=f32)

    def mm(a, w):
        return dot(a, w, (((1,), (0,)), ((), ())))

    def bdot(a, b2, ca, cb):
        return dot(a, b2, (((ca,), (cb,)), ((0,), (0,))))

    riota = lax.broadcasted_iota(jnp.int32, (R, 1), 0)
    jmask = (riota % JP) < J

    S1 = jax.nn.relu(mm(X, sk1_W) + sk1_b)
    SK = jax.nn.relu(mm(S1, sk2_W) + sk2_b)
    SKm = jnp.where(jmask, SK, 0.0)
    SK3 = SKm.reshape(TBt, JP, H)
    smean = jnp.sum(SK3, axis=1) * (1.0 / J)
    se = jax.nn.sigmoid(mm(jax.nn.relu(mm(smean, se_W1) + se_b1), se_W2) + se_b2)
    TI3 = SK3 * se[:, None, :]
    TI = TI3.reshape(R, H)

    QKV = mm(TI, WqkvT) + bqkv
    QKV3 = QKV.reshape(TBt, JP, 3 * H)
    cmask = lax.broadcasted_iota(jnp.int32, (TBt, JP, JP), 2) < J
    aos = []
    for h in range(NH):
        q = QKV3[:, :, h * DH:(h + 1) * DH]
        k = QKV3[:, :, H + h * DH:H + (h + 1) * DH]
        v = QKV3[:, :, 2 * H + h * DH:2 * H + (h + 1) * DH]
        sc = bdot(q, k, 2, 2) * 0.25
        sc = jnp.where(cmask, sc, NEG)
        m = jnp.max(sc, axis=-1, keepdims=True)
        e = jnp.exp(sc - m)
        att = e / jnp.sum(e, axis=-1, keepdims=True)
        aos.append(bdot(att, v, 2, 1))
    AO3 = jnp.concatenate(aos, axis=2)
    AO = mm(AO3.reshape(R, H), WoT) + bo
    h1 = _ln(TI + AO, ln1_g, ln1_b)
    ff = mm(jax.nn.relu(mm(h1, ff_W1) + ff_b1), ff_W2) + ff_b2
    to = _ln(h1 + ff, ln2_g, ln2_b)
    to3 = to.reshape(TBt, JP, H)

    pf = jnp.zeros((TBt, 2 * H), f32)
    for j in range(J):
        pf = pf + mm(to3[:, j, :], Wsk_fu[j * H:(j + 1) * H, :])

    P1 = jax.nn.relu(mm(X, sp1_W) + sp1_b)
    SP = jax.nn.relu(mm(P1, sp2_W) + sp2_b)
    SPm = jnp.where(jmask, SP, 0.0)
    spf = jnp.sum(SPm.reshape(TBt, JP, H), axis=1) * (1.0 / J)
    pf = pf + mm(spf, Wsp_fu)

    X3 = X.reshape(TBt, JP, Fd)
    G = mm(mah_L, mah_L.transpose())
    XG = mm(X, G)
    qdm = jnp.sum(X * XG, axis=1).reshape(TBt, JP)
    qde = jnp.sum(X * X, axis=1).reshape(TBt, JP)
    crm = bdot(X3, XG.reshape(TBt, JP, Fd), 2, 2)
    cre = bdot(X3, X3, 2, 2)
    md2 = qdm[:, :, None] + qdm[:, None, :] - 2.0 * crm
    md = jnp.sqrt(jnp.maximum(md2, 0.0) + 1e-12)
    d2 = qde[:, :, None] + qde[:, None, :] - 2.0 * cre
    dist = jnp.sqrt(jnp.maximum(d2, 0.0) + 1e-12)
    dist = jnp.where(cmask, dist, 1e30)

    j1 = jnp.zeros((TBt, H), f32)
    for i in range(J):
        j1 = j1 + mm(md[:, i, :], Wfull_jcd[i * JP:(i + 1) * JP, :])
    J1 = jax.nn.relu(j1 + jcd1_b)
    J2 = jax.nn.relu(mm(J1, jcd2_W) + jcd2_b)
    pf = pf + mm(J2, Wjcd_fu)
    pf_ref[0, 0] = pf

    iota2 = lax.broadcasted_iota(jnp.int32, (TBt, JP, JP), 2)
    d = dist
    ds = []
    for t in range(5):
        m = jnp.min(d, axis=-1, keepdims=True)
        ds.append(m[:, :, 0])
        cand = jnp.where(d == m, iota2, 1000)
        jmin = jnp.min(cand, axis=-1, keepdims=True)
        d = jnp.where(iota2 == jmin, 1e30, d)
    d2_, d3_, d4_, d5_ = ds[1], ds[2], ds[3], ds[4]
    m2 = (d2_ + d3_) * 0.5
    sd2 = jnp.sqrt((d2_ - m2) ** 2 + (d3_ - m2) ** 2)
    m3 = (d2_ + d3_ + d4_) * (1.0 / 3.0)
    sd3 = jnp.sqrt(((d2_ - m3) ** 2 + (d3_ - m3) ** 2 + (d4_ - m3) ** 2) * 0.5)
    m4 = (d2_ + d3_ + d4_ + d5_) * 0.25
    sd4 = jnp.sqrt(((d2_ - m4) ** 2 + (d3_ - m4) ** 2 +
                    (d4_ - m4) ** 2 + (d5_ - m4) ** 2) * (1.0 / 3.0))
    part = jnp.stack([m2, sd2, d2_, m3, sd3, d2_, m4, sd4, d2_], axis=1)
    psum = jnp.sum(part, axis=0)

    @pl.when(pl.program_id(1) == 0)
    def _init():
        topo9_ref[0] = psum

    @pl.when(pl.program_id(1) > 0)
    def _acc():
        topo9_ref[0] += psum


def _kernel_c(topo9_ref, xt_ref, eiT_ref,
              topo1_W, topo1_b, topo2_W, topo2_b,
              stat1_W, stat1_b, stat2_W, stat2_b,
              Wtopo_fu, Wst_fu,
              pf2_ref):
    (topo1_W, topo1_b, topo2_W, topo2_b, stat1_W, stat1_b,
     stat2_W, stat2_b, Wtopo_fu, Wst_fu) = [
        r[...] for r in (topo1_W, topo1_b, topo2_W, topo2_b, stat1_W,
                         stat1_b, stat2_W, stat2_b, Wtopo_fu, Wst_fu)]
    f32 = jnp.float32
    NN = B * JP
    dot = functools.partial(lax.dot_general, preferred_element_type=f32)

    def mm(a, w):
        return dot(a, w, (((1,), (0,)), ((), ())))

    ei = eiT_ref[...] % J
    lane = lax.broadcasted_iota(jnp.int32, (48, JP), 1)
    OHs = (lane == ei[:, 0:1]).astype(f32)
    OHd = (lane == ei[:, 1:2]).astype(f32)
    C = dot(OHd, OHs, (((0,), (0,)), ((), ())))
    deg = 1.0 + jnp.sum(C, axis=1, keepdims=True)
    dinv = jax.lax.rsqrt(deg)
    dinv_r = dinv.reshape(1, JP)
    eye = (lax.broadcasted_iota(jnp.int32, (JP, JP), 0) ==
           lax.broadcasted_iota(jnp.int32, (JP, JP), 1)).astype(f32)
    M = (C + eye) * dinv * dinv_r

    riota = lax.broadcasted_iota(jnp.int32, (NN, 1), 0)
    jmask = (riota % JP) < J

    def gcn(hin, W, b):
        h = mm(hin, W)
        base = h + b
        mixed = mm(M, h[0:JP, :]) + b
        return jax.nn.relu(jnp.concatenate([mixed, base[JP:, :]], axis=0))

    t9sum = topo9_ref[...] * (1.0 / T)
    tn = jnp.swapaxes(t9sum, 1, 2).reshape(NN, 9)
    t1 = gcn(tn, topo1_W, topo1_b)
    t2 = gcn(t1, topo2_W, topo2_b)
    t2m = jnp.where(jmask, t2, 0.0).reshape(B, JP, H)
    topof = jnp.sum(t2m, axis=1) * (1.0 / J)

    xt = xt_ref[...]
    S1 = jnp.sum(xt, axis=1, keepdims=True)
    mean = S1 * (1.0 / T)
    cen = xt - mean
    c2 = cen * cen
    S2 = jnp.sum(c2, axis=1, keepdims=True)
    S3 = jnp.sum(c2 * cen, axis=1, keepdims=True)
    S4 = jnp.sum(c2 * c2, axis=1, keepdims=True)
    mn = jnp.min(xt, axis=1, keepdims=True)
    mx = jnp.max(xt, axis=1, keepdims=True)
    var = S2 * (1.0 / (T - 1)) + 1e-10
    sdv = jnp.sqrt(S2 * (1.0 / (T - 1)))
    sq = jnp.sqrt(var)
    skw = (S3 * (1.0 / T)) / (sq * sq * sq + 1e-10)
    kur = (S4 * (1.0 / T)) / (var * var + 1e-10) - 3.0
    stats6 = jnp.concatenate([mean, sdv, mn, mx, skw, kur], axis=1)
    ST = jnp.sum(stats6.reshape(NN, Fd, 6), axis=1) * (1.0 / Fd)
    s1 = gcn(ST, stat1_W, stat1_b)
    s2 = gcn(s1, stat2_W, stat2_b)
    s2m = jnp.where(jmask, s2, 0.0)
    stf = jnp.sum(s2m, axis=0, keepdims=True) * (1.0 / (B * J))

    pf2_ref[...] = mm(topof, Wtopo_fu) + mm(stf, Wst_fu)


def _kernel_d(pf_ref, pf2_ref,
              fu1_b, fln_g, fln_b, fu2_W, fu2_b,
              g0_WihT, g0_WhhT, g0_bih, g0_bhh,
              g1_WihT, g1_WhhT, g1_bih, g1_bhh,
              fc_W, fc_b,
              out_ref, gi_ref, ys_ref):
    (fu1_b, fln_g, fln_b, fu2_W, fu2_b,
     g0_WihT, g0_WhhT, g0_bih, g0_bhh,
     g1_WihT, g1_WhhT, g1_bih, g1_bhh, fc_W, fc_b) = [
        r[...] for r in (fu1_b, fln_g, fln_b, fu2_W, fu2_b,
                         g0_WihT, g0_WhhT, g0_bih, g0_bhh,
                         g1_WihT, g1_WhhT, g1_bih, g1_bhh, fc_W, fc_b)]
    f32 = jnp.float32
    dot = functools.partial(lax.dot_general, preferred_element_type=f32)

    def mm(a, w):
        return dot(a, w, (((1,), (0,)), ((), ())))

    Fpre = pf_ref[...] + pf2_ref[...][None, :, :] + fu1_b
    Ff = jax.nn.relu(_ln(Fpre, fln_g, fln_b)).reshape(T * B, 2 * H)
    GRUin = mm(Ff, fu2_W) + fu2_b

    def layer(xflat, WihT, WhhT, bih, bhh, store_ys):
        gi_ref[...] = (mm(xflat, WihT) + bih).reshape(T, B, 3 * H)

        def body(t, hh):
            gi = gi_ref[t]
            gh = mm(hh, WhhT) + bhh
            r = jax.nn.sigmoid(gi[:, 0:H] + gh[:, 0:H])
            z = jax.nn.sigmoid(gi[:, H:2 * H] + gh[:, H:2 * H])
            n = jnp.tanh(gi[:, 2 * H:] + r * gh[:, 2 * H:])
            h2 = (1.0 - z) * n + z * hh
            if store_ys:
                ys_ref[t] = h2
            return h2

        return lax.fori_loop(0, T, body, jnp.zeros((B, H), f32))

    layer(GRUin, g0_WihT, g0_WhhT, g0_bih, g0_bhh, True)
    h_last = layer(ys_ref[...].reshape(T * B, H),
                   g1_WihT, g1_WhhT, g1_bih, g1_bhh, False)
    out_ref[...] = mm(h_last, fc_W) + fc_b


def _full(shape):
    nd = len(shape)
    return pl.BlockSpec(shape, lambda b, s: (0,) * nd)


def kernel(x, edge_index, params):
    p = params
    f32 = jnp.float32
    xpad = jnp.pad(x, ((0, 0), (0, 0), (0, JP - J), (0, 0)))
    xp = xpad.reshape(B, T * JP, Fd)
    xt = xpad.transpose(0, 2, 3, 1).reshape(B * JP * Fd, T)
    eiT = edge_index.astype(jnp.int32).transpose()

    iu, ju = np.triu_indices(J, 1)
    Wfull_jcd = (jnp.zeros((J, JP, H), f32)
                 .at[iu, ju].set(0.5 * p['jcd1_W'])
                 .at[ju, iu].set(0.5 * p['jcd1_W'])).reshape(J * JP, H)

    fu1 = p['fu1_W']
    Wsk_fu = fu1[:J * H]
    Wjcd_fu = fu1[J * H:J * H + H]
    Wsp_fu = fu1[J * H + H:J * H + 2 * H]
    Wtopo_fu = fu1[J * H + 2 * H:J * H + 3 * H]
    Wst_fu = fu1[J * H + 3 * H:]

    def row(v):
        return v.reshape(1, -1)

    pf, topo9 = pl.pallas_call(
        _kernel_a,
        grid=(B, SPB),
        in_specs=[pl.BlockSpec((1, TBt * JP, Fd), lambda b, s: (b, s, 0))] +
                 [_full(w.shape) for w in (
                     p['sk1_W'], row(p['sk1_b']), p['sk2_W'], row(p['sk2_b']),
                     p['se_W1'], row(p['se_b1']), p['se_W2'], row(p['se_b2']),
                     p['attn_Wqkv'].transpose(), row(p['attn_bqkv']),
                     p['attn_Wo'].transpose(), row(p['attn_bo']),
                     row(p['ln1_g']), row(p['ln1_b']), row(p['ln2_g']), row(p['ln2_b']),
                     p['ff_W1'], row(p['ff_b1']), p['ff_W2'], row(p['ff_b2']),
                     p['sp1_W'], row(p['sp1_b']), p['sp2_W'], row(p['sp2_b']),
                     p['mah_L'],
                     Wsk_fu, Wfull_jcd, row(p['jcd1_b']), p['jcd2_W'],
                     row(p['jcd2_b']), Wjcd_fu, Wsp_fu)],
        out_specs=[pl.BlockSpec((1, 1, TBt, 2 * H), lambda b, s: (b, s, 0, 0)),
                   pl.BlockSpec((1, 9, JP), lambda b, s: (b, 0, 0))],
        out_shape=[jax.ShapeDtypeStruct((B, SPB, TBt, 2 * H), f32),
                   jax.ShapeDtypeStruct((B, 9, JP), f32)],
    )(xp,
      p['sk1_W'], row(p['sk1_b']), p['sk2_W'], row(p['sk2_b']),
      p['se_W1'], row(p['se_b1']), p['se_W2'], row(p['se_b2']),
      p['attn_Wqkv'].transpose(), row(p['attn_bqkv']),
      p['attn_Wo'].transpose(), row(p['attn_bo']),
      row(p['ln1_g']), row(p['ln1_b']), row(p['ln2_g']), row(p['ln2_b']),
      p['ff_W1'], row(p['ff_b1']), p['ff_W2'], row(p['ff_b2']),
      p['sp1_W'], row(p['sp1_b']), p['sp2_W'], row(p['sp2_b']),
      p['mah_L'],
      Wsk_fu, Wfull_jcd, row(p['jcd1_b']), p['jcd2_W'],
      row(p['jcd2_b']), Wjcd_fu, Wsp_fu)

    pf2 = pl.pallas_call(
        _kernel_c,
        out_shape=jax.ShapeDtypeStruct((B, 2 * H), f32),
    )(topo9, xt, eiT,
      p['topo1_W'], row(p['topo1_b']), p['topo2_W'], row(p['topo2_b']),
      p['stat1_W'], row(p['stat1_b']), p['stat2_W'], row(p['stat2_b']),
      Wtopo_fu, Wst_fu)

    pf_t = pf.reshape(B, T, 2 * H).transpose(1, 0, 2)
    out = pl.pallas_call(
        _kernel_d,
        out_shape=jax.ShapeDtypeStruct((B, OUT), f32),
        scratch_shapes=[pltpu.VMEM((T, B, 3 * H), f32),
                        pltpu.VMEM((T, B, H), f32)],
    )(pf_t, pf2,
      row(p['fu1_b']), row(p['fln_g']), row(p['fln_b']), p['fu2_W'], row(p['fu2_b']),
      p['gru0_Wih'].transpose(), p['gru0_Whh'].transpose(),
      row(p['gru0_bih']), row(p['gru0_bhh']),
      p['gru1_Wih'].transpose(), p['gru1_Whh'].transpose(),
      row(p['gru1_bih']), row(p['gru1_bhh']),
      p['fc_W'], row(p['fc_b']))
    return out
